# baseline (speedup 1.0000x reference)
import functools

import jax
import jax.numpy as jnp
from jax import lax
from jax.experimental import pallas as pl
from jax.experimental.pallas import tpu as pltpu
from jax.experimental.pallas import tpu_sc as plsc

N = 10000
E = 320000
D = 128
H = 64
G = 128
T = 10
L = 4

RB = 1000
NB = N // RB
LW = 128
NF = N // 2
RBF = 1000
NBF = NF // RBF

NC = 2
NS = 16
NW = NC * NS
EPW = E // NW
CK = 125
NCH = EPW // CK
NP = 10240
RPT = NP // NS

_HIGH = lax.Precision.DEFAULT


def _dot(a, b):
    return jnp.dot(a, b, preferred_element_type=jnp.float32, precision=_HIGH)


def _first_body(x_ref, w1_ref, b1_ref, w2_ref, b2_ref, h_ref):
    a = jnp.maximum(_dot(x_ref[...], w1_ref[...]) + b1_ref[...], 0.0)
    h_ref[...] = jnp.maximum(_dot(a, w2_ref[...]) + b2_ref[...], 0.0)


def _pool_body(h_ref, bt_ref, bb_ref, lw_ref, prev_ref, lbc_ref, lba_ref,
               out_ref):
    oht = _fold_onehot(bt_ref)
    ohb = _fold_onehot(bb_ref)
    p = _dot(oht, h_ref[...])
    q = _dot(ohb, h_ref[...])
    lo = (lax.broadcasted_iota(jnp.int32, (1, LW), 1) < H).astype(jnp.float32)
    csum = (jnp.sum(oht, axis=1, keepdims=True)
            + jnp.sum(ohb, axis=1, keepdims=True))
    o = _dot(p * lo + q * (1.0 - lo), lw_ref[...]) + csum * lbc_ref[...]

    @pl.when(pl.program_id(0) == 0)
    def _():
        out_ref[...] = prev_ref[...] + lba_ref[...] + o

    @pl.when(pl.program_id(0) != 0)
    def _():
        out_ref[...] += o


def _fold_onehot(bt_ref):
    ids = jnp.broadcast_to(jnp.reshape(bt_ref[...], (1, RBF)), (G, RBF))
    return (ids == lax.broadcasted_iota(jnp.int32, (G, RBF), 0)).astype(jnp.float32)


def _layer_body(h_ref, agg_ref, eps_ref, w1_ref, b1_ref, w2_ref, b2_ref,
                hn_ref):
    z = h_ref[...] * eps_ref[...] + agg_ref[0] + agg_ref[1]
    z = jnp.maximum(_dot(z, w1_ref[...]) + b1_ref[...], 0.0)
    hn_ref[...] = jnp.maximum(_dot(z, w2_ref[...]) + b2_ref[...], 0.0)


_first_call = pl.pallas_call(
    _first_body,
    grid=(NB,),
    in_specs=[
        pl.BlockSpec((RB, D), lambda i: (i, 0)),
        pl.BlockSpec((D, H), lambda i: (0, 0)),
        pl.BlockSpec((1, H), lambda i: (0, 0)),
        pl.BlockSpec((H, H), lambda i: (0, 0)),
        pl.BlockSpec((1, H), lambda i: (0, 0)),
    ],
    out_specs=pl.BlockSpec((RB, H), lambda i: (i, 0)),
    out_shape=jax.ShapeDtypeStruct((N, H), jnp.float32),
)

_pool_call = pl.pallas_call(
    _pool_body,
    grid=(NBF,),
    in_specs=[
        pl.BlockSpec((RBF, LW), lambda i: (i, 0)),
        pl.BlockSpec((1, 1, RBF), lambda i: (i, 0, 0)),
        pl.BlockSpec((1, 1, RBF), lambda i: (i, 0, 0)),
        pl.BlockSpec((LW, LW), lambda i: (0, 0)),
        pl.BlockSpec((G, LW), lambda i: (0, 0)),
        pl.BlockSpec((1, LW), lambda i: (0, 0)),
        pl.BlockSpec((1, LW), lambda i: (0, 0)),
    ],
    out_specs=pl.BlockSpec((G, LW), lambda i: (0, 0)),
    out_shape=jax.ShapeDtypeStruct((G, LW), jnp.float32),
)

_layer_call = pl.pallas_call(
    _layer_body,
    grid=(NBF,),
    in_specs=[
        pl.BlockSpec((RBF, LW), lambda i: (i, 0)),
        pl.BlockSpec((NC, RBF, LW), lambda i: (0, i, 0)),
        pl.BlockSpec((1, LW), lambda i: (0, 0)),
        pl.BlockSpec((LW, LW), lambda i: (0, 0)),
        pl.BlockSpec((1, LW), lambda i: (0, 0)),
        pl.BlockSpec((LW, LW), lambda i: (0, 0)),
        pl.BlockSpec((1, LW), lambda i: (0, 0)),
    ],
    out_specs=pl.BlockSpec((RBF, LW), lambda i: (i, 0)),
    out_shape=jax.ShapeDtypeStruct((NF, LW), jnp.float32),
)


SC_J = 4


def _sc_agg_body(h_hbm, edge_hbm, z_hbm, out_hbm,
                 srcbuf, dstbuf, rows, acc, gsem, ssem, psem):
    c = lax.axis_index("c")
    s = lax.axis_index("s")
    w = c * NS + s
    ca = pltpu.async_copy(edge_hbm.at[0, w], srcbuf, psem.at[0])
    cb = pltpu.async_copy(edge_hbm.at[1, w], dstbuf, psem.at[1])
    cc = pltpu.async_copy(z_hbm, acc.at[pl.ds(s * RPT, RPT)], psem.at[2])

    def fire_gather(k, j):
        pltpu.async_copy(h_hbm.at[srcbuf.at[k]], rows.at[j], gsem.at[j])

    def wait_gather(j):
        pltpu.make_async_copy(h_hbm.at[srcbuf.at[0]], rows.at[j],
                              gsem.at[j]).wait()

    def fire_scatter(k, j):
        pltpu.async_copy(rows.at[j], acc.at[dstbuf.at[k]], ssem.at[j],
                         add=True)

    def wait_scatter(j):
        pltpu.make_async_copy(rows.at[0], acc.at[dstbuf.at[0]],
                              ssem.at[j]).wait()

    ca.wait()
    for j in range(SC_J):
        fire_gather(j, j)
    cb.wait()
    cc.wait()
    plsc.subcore_barrier()

    @pl.loop(0, NCH // (2 * SC_J))
    def _grp(gg):
        base = gg * 2 * SC_J
        for j in range(SC_J):
            @pl.when(gg >= 1)
            def _():
                wait_scatter(SC_J + j)

            fire_gather(base + SC_J + j, SC_J + j)
        for j in range(SC_J):
            wait_gather(j)
            fire_scatter(base + j, j)

        @pl.when(gg + 1 < NCH // (2 * SC_J))
        def _():
            for j in range(SC_J):
                wait_scatter(j)
                fire_gather(base + 2 * SC_J + j, j)

        for j in range(SC_J):
            wait_gather(SC_J + j)
            fire_scatter(base + SC_J + j, SC_J + j)

    for j in range(2 * SC_J):
        wait_scatter(j)

    plsc.subcore_barrier()
    pltpu.sync_copy(acc.at[pl.ds(s * RPT, RPT)],
                    out_hbm.at[c, pl.ds(s * RPT, RPT)])


@functools.lru_cache(maxsize=None)
def _sc_agg():
  return pl.kernel(
    _sc_agg_body,
    out_type=jax.ShapeDtypeStruct((NC, NP, H), jnp.float32),
    mesh=plsc.VectorSubcoreMesh(core_axis_name="c", subcore_axis_name="s",
                                num_cores=NC, num_subcores=NS),
    scratch_types=[
        pltpu.VMEM((NCH, CK), jnp.int32),
        pltpu.VMEM((NCH, CK), jnp.int32),
        pltpu.VMEM((2 * SC_J, CK, H), jnp.float32),
        pltpu.VMEM_SHARED((NP, H), jnp.float32),
        pltpu.SemaphoreType.DMA((2 * SC_J,)),
        pltpu.SemaphoreType.DMA((2 * SC_J,)),
        pltpu.SemaphoreType.DMA((3,)),
    ],
    compiler_params=pltpu.CompilerParams(use_tc_tiling_on_sc=False),
  )


def _fold(w, b, g, be, m, v):
    inv = g / jnp.sqrt(v + 1e-5)
    return w * inv[None, :], (b - m) * inv + be


def kernel(x, edge_index, batch, fW1, fb1, fg1, fbe1, fm1, fv1, fW2, fb2, fg2,
           fbe2, fm2, fv2, eps, cW1, cb1, cg1, cbe1, cm1, cv1, cW2, cb2, cg2,
           cbe2, cm2, cv2, linW, linb):
    fW1e, fb1e = _fold(fW1, fb1, fg1, fbe1, fm1, fv1)
    fW2e, fb2e = _fold(fW2, fb2, fg2, fbe2, fm2, fv2)
    inv1 = cg1 / jnp.sqrt(cv1 + 1e-5)
    cW1e = cW1 * inv1[:, None, :]
    cb1e = (cb1 - cm1) * inv1 + cbe1
    inv2 = cg2 / jnp.sqrt(cv2 + 1e-5)
    cW2e = cW2 * inv2[:, None, :]
    cb2e = (cb2 - cm2) * inv2 + cbe2

    linWp = jnp.zeros((L + 1, H, LW), jnp.float32).at[:, :, :T].set(linW)
    bt3 = batch[0::2].reshape(NBF, 1, RBF)
    bb3 = batch[1::2].reshape(NBF, 1, RBF)
    edge4 = edge_index.reshape(2, NW, NCH, CK)
    zblk = jnp.zeros((RPT, H), jnp.float32)
    epsf = jnp.broadcast_to((1.0 + eps)[:, None, None], (L, 1, LW))

    zpad = jnp.zeros((L, LW, LW), jnp.float32)
    cW1f = zpad.at[:, :H, :H].set(cW1e).at[:, H:, H:].set(cW1e)
    cW2f = zpad.at[:, :H, :H].set(cW2e).at[:, H:, H:].set(cW2e)
    cb1f = jnp.tile(cb1e, (1, 2))
    cb2f = jnp.tile(cb2e, (1, 2))
    linWf = jnp.concatenate([linWp, linWp], axis=1)

    linbp = jnp.zeros((L + 1, 1, LW), jnp.float32).at[:, 0, :T].set(linb)
    zrow = jnp.zeros((1, LW), jnp.float32)
    zout = jnp.zeros((G, LW), jnp.float32)

    h = _first_call(x, fW1e, fb1e[None], fW2e, fb2e[None])
    hf = h.reshape(NF, LW)
    out = _pool_call(hf, bt3, bb3, linWf[0], zout, linbp[0], zrow)
    for l in range(L):
        agg = _sc_agg()(hf.reshape(N, H), edge4, zblk)
        aggf = agg.reshape(NC, NP // 2, LW)
        hf = _layer_call(hf, aggf, epsf[l], cW1f[l], cb1f[l][None],
                         cW2f[l], cb2f[l][None])
        out = _pool_call(hf, bt3, bb3, linWf[l + 1], out, zrow, linbp[l + 1])
    return out[:, :T]


# speedup vs baseline: 1.0105x; 1.0105x over previous
import functools

import jax
import jax.numpy as jnp
from jax import lax
from jax.experimental import pallas as pl
from jax.experimental.pallas import tpu as pltpu
from jax.experimental.pallas import tpu_sc as plsc

N = 10000
E = 320000
D = 128
H = 64
G = 128
T = 10
L = 4

RB = 1000
NB = N // RB
LW = 128
NF = N // 2
RBF = 1000
NBF = NF // RBF

NC = 2
NS = 16
NW = NC * NS
EPW = E // NW
CK = 100
NCH = EPW // CK
NP = 10240
RPT = NP // NS

_HIGH = lax.Precision.DEFAULT


def _dot(a, b):
    return jnp.dot(a, b, preferred_element_type=jnp.float32, precision=_HIGH)


def _first_body(x_ref, w1_ref, b1_ref, w2_ref, b2_ref, h_ref):
    a = jnp.maximum(_dot(x_ref[...], w1_ref[...]) + b1_ref[...], 0.0)
    h_ref[...] = jnp.maximum(_dot(a, w2_ref[...]) + b2_ref[...], 0.0)


def _pool_body(h_ref, bt_ref, bb_ref, lw_ref, prev_ref, lbc_ref, lba_ref,
               out_ref):
    oht = _fold_onehot(bt_ref)
    ohb = _fold_onehot(bb_ref)
    p = _dot(oht, h_ref[...])
    q = _dot(ohb, h_ref[...])
    lo = (lax.broadcasted_iota(jnp.int32, (1, LW), 1) < H).astype(jnp.float32)
    csum = (jnp.sum(oht, axis=1, keepdims=True)
            + jnp.sum(ohb, axis=1, keepdims=True))
    o = _dot(p * lo + q * (1.0 - lo), lw_ref[...]) + csum * lbc_ref[...]

    @pl.when(pl.program_id(0) == 0)
    def _():
        out_ref[...] = prev_ref[...] + lba_ref[...] + o

    @pl.when(pl.program_id(0) != 0)
    def _():
        out_ref[...] += o


def _fold_onehot(bt_ref):
    ids = jnp.broadcast_to(jnp.reshape(bt_ref[...], (1, RBF)), (G, RBF))
    return (ids == lax.broadcasted_iota(jnp.int32, (G, RBF), 0)).astype(jnp.float32)


def _layer_body(h_ref, agg_ref, eps_ref, w1_ref, b1_ref, w2_ref, b2_ref,
                hn_ref):
    z = h_ref[...] * eps_ref[...] + agg_ref[0] + agg_ref[1]
    z = jnp.maximum(_dot(z, w1_ref[...]) + b1_ref[...], 0.0)
    hn_ref[...] = jnp.maximum(_dot(z, w2_ref[...]) + b2_ref[...], 0.0)


_first_call = pl.pallas_call(
    _first_body,
    grid=(NB,),
    in_specs=[
        pl.BlockSpec((RB, D), lambda i: (i, 0)),
        pl.BlockSpec((D, H), lambda i: (0, 0)),
        pl.BlockSpec((1, H), lambda i: (0, 0)),
        pl.BlockSpec((H, H), lambda i: (0, 0)),
        pl.BlockSpec((1, H), lambda i: (0, 0)),
    ],
    out_specs=pl.BlockSpec((RB, H), lambda i: (i, 0)),
    out_shape=jax.ShapeDtypeStruct((N, H), jnp.float32),
)

_pool_call = pl.pallas_call(
    _pool_body,
    grid=(NBF,),
    in_specs=[
        pl.BlockSpec((RBF, LW), lambda i: (i, 0)),
        pl.BlockSpec((1, 1, RBF), lambda i: (i, 0, 0)),
        pl.BlockSpec((1, 1, RBF), lambda i: (i, 0, 0)),
        pl.BlockSpec((LW, LW), lambda i: (0, 0)),
        pl.BlockSpec((G, LW), lambda i: (0, 0)),
        pl.BlockSpec((1, LW), lambda i: (0, 0)),
        pl.BlockSpec((1, LW), lambda i: (0, 0)),
    ],
    out_specs=pl.BlockSpec((G, LW), lambda i: (0, 0)),
    out_shape=jax.ShapeDtypeStruct((G, LW), jnp.float32),
)

_layer_call = pl.pallas_call(
    _layer_body,
    grid=(NBF,),
    in_specs=[
        pl.BlockSpec((RBF, LW), lambda i: (i, 0)),
        pl.BlockSpec((NC, RBF, LW), lambda i: (0, i, 0)),
        pl.BlockSpec((1, LW), lambda i: (0, 0)),
        pl.BlockSpec((LW, LW), lambda i: (0, 0)),
        pl.BlockSpec((1, LW), lambda i: (0, 0)),
        pl.BlockSpec((LW, LW), lambda i: (0, 0)),
        pl.BlockSpec((1, LW), lambda i: (0, 0)),
    ],
    out_specs=pl.BlockSpec((RBF, LW), lambda i: (i, 0)),
    out_shape=jax.ShapeDtypeStruct((NF, LW), jnp.float32),
)


SC_J = 5


def _sc_agg_body(h_hbm, edge_hbm, z_hbm, out_hbm,
                 srcbuf, dstbuf, rows, acc, gsem, ssem, psem):
    c = lax.axis_index("c")
    s = lax.axis_index("s")
    w = c * NS + s
    ca = pltpu.async_copy(edge_hbm.at[0, w], srcbuf, psem.at[0])
    cb = pltpu.async_copy(edge_hbm.at[1, w], dstbuf, psem.at[1])
    cc = pltpu.async_copy(z_hbm, acc.at[pl.ds(s * RPT, RPT)], psem.at[2])

    def fire_gather(k, j):
        pltpu.async_copy(h_hbm.at[srcbuf.at[k]], rows.at[j], gsem.at[j])

    def wait_gather(j):
        pltpu.make_async_copy(h_hbm.at[srcbuf.at[0]], rows.at[j],
                              gsem.at[j]).wait()

    def fire_scatter(k, j):
        pltpu.async_copy(rows.at[j], acc.at[dstbuf.at[k]], ssem.at[j],
                         add=True)

    def wait_scatter(j):
        pltpu.make_async_copy(rows.at[0], acc.at[dstbuf.at[0]],
                              ssem.at[j]).wait()

    ca.wait()
    for j in range(SC_J):
        fire_gather(j, j)
    cb.wait()
    cc.wait()
    plsc.subcore_barrier()

    @pl.loop(0, NCH // (2 * SC_J))
    def _grp(gg):
        base = gg * 2 * SC_J
        for j in range(SC_J):
            @pl.when(gg >= 1)
            def _():
                wait_scatter(SC_J + j)

            fire_gather(base + SC_J + j, SC_J + j)
        for j in range(SC_J):
            wait_gather(j)
            fire_scatter(base + j, j)

        @pl.when(gg + 1 < NCH // (2 * SC_J))
        def _():
            for j in range(SC_J):
                wait_scatter(j)
                fire_gather(base + 2 * SC_J + j, j)

        for j in range(SC_J):
            wait_gather(SC_J + j)
            fire_scatter(base + SC_J + j, SC_J + j)

    for j in range(2 * SC_J):
        wait_scatter(j)

    plsc.subcore_barrier()
    pltpu.sync_copy(acc.at[pl.ds(s * RPT, RPT)],
                    out_hbm.at[c, pl.ds(s * RPT, RPT)])


@functools.lru_cache(maxsize=None)
def _sc_agg():
  return pl.kernel(
    _sc_agg_body,
    out_type=jax.ShapeDtypeStruct((NC, NP, H), jnp.float32),
    mesh=plsc.VectorSubcoreMesh(core_axis_name="c", subcore_axis_name="s",
                                num_cores=NC, num_subcores=NS),
    scratch_types=[
        pltpu.VMEM((NCH, CK), jnp.int32),
        pltpu.VMEM((NCH, CK), jnp.int32),
        pltpu.VMEM((2 * SC_J, CK, H), jnp.float32),
        pltpu.VMEM_SHARED((NP, H), jnp.float32),
        pltpu.SemaphoreType.DMA((2 * SC_J,)),
        pltpu.SemaphoreType.DMA((2 * SC_J,)),
        pltpu.SemaphoreType.DMA((3,)),
    ],
    compiler_params=pltpu.CompilerParams(use_tc_tiling_on_sc=False),
  )


def _fold(w, b, g, be, m, v):
    inv = g / jnp.sqrt(v + 1e-5)
    return w * inv[None, :], (b - m) * inv + be


def kernel(x, edge_index, batch, fW1, fb1, fg1, fbe1, fm1, fv1, fW2, fb2, fg2,
           fbe2, fm2, fv2, eps, cW1, cb1, cg1, cbe1, cm1, cv1, cW2, cb2, cg2,
           cbe2, cm2, cv2, linW, linb):
    fW1e, fb1e = _fold(fW1, fb1, fg1, fbe1, fm1, fv1)
    fW2e, fb2e = _fold(fW2, fb2, fg2, fbe2, fm2, fv2)
    inv1 = cg1 / jnp.sqrt(cv1 + 1e-5)
    cW1e = cW1 * inv1[:, None, :]
    cb1e = (cb1 - cm1) * inv1 + cbe1
    inv2 = cg2 / jnp.sqrt(cv2 + 1e-5)
    cW2e = cW2 * inv2[:, None, :]
    cb2e = (cb2 - cm2) * inv2 + cbe2

    linWp = jnp.zeros((L + 1, H, LW), jnp.float32).at[:, :, :T].set(linW)
    bt3 = batch[0::2].reshape(NBF, 1, RBF)
    bb3 = batch[1::2].reshape(NBF, 1, RBF)
    edge4 = edge_index.reshape(2, NW, NCH, CK)
    zblk = jnp.zeros((RPT, H), jnp.float32)
    epsf = jnp.broadcast_to((1.0 + eps)[:, None, None], (L, 1, LW))

    zpad = jnp.zeros((L, LW, LW), jnp.float32)
    cW1f = zpad.at[:, :H, :H].set(cW1e).at[:, H:, H:].set(cW1e)
    cW2f = zpad.at[:, :H, :H].set(cW2e).at[:, H:, H:].set(cW2e)
    cb1f = jnp.tile(cb1e, (1, 2))
    cb2f = jnp.tile(cb2e, (1, 2))
    linWf = jnp.concatenate([linWp, linWp], axis=1)

    linbp = jnp.zeros((L + 1, 1, LW), jnp.float32).at[:, 0, :T].set(linb)
    zrow = jnp.zeros((1, LW), jnp.float32)
    zout = jnp.zeros((G, LW), jnp.float32)

    h = _first_call(x, fW1e, fb1e[None], fW2e, fb2e[None])
    hf = h.reshape(NF, LW)
    out = _pool_call(hf, bt3, bb3, linWf[0], zout, linbp[0], zrow)
    for l in range(L):
        agg = _sc_agg()(hf.reshape(N, H), edge4, zblk)
        aggf = agg.reshape(NC, NP // 2, LW)
        hf = _layer_call(hf, aggf, epsf[l], cW1f[l], cb1f[l][None],
                         cW2f[l], cb2f[l][None])
        out = _pool_call(hf, bt3, bb3, linWf[l + 1], out, zrow, linbp[l + 1])
    return out[:, :T]
